# baseline (speedup 1.0000x reference)
import jax
import jax.numpy as jnp
import numpy as np
from jax.experimental import pallas as pl

B, N, KC, T, D, DH, P = 128, 64, 2, 32, 64, 64, 8
D2 = D // 2
G = 4
T2 = T // 2
NT2 = N * T2
TP = T * P
TWO_PI = np.float32(2.0 * np.pi)
PI = np.float32(np.pi)
HALF_PI = np.float32(np.pi / 2)
QUARTER_PI = np.float32(np.pi / 4)
INV_BUCKET = np.float32(P / (2.0 * np.pi))

_ROW_T = np.arange(TP)[:, None] // P
_COL_T2 = np.arange(NT2)[None, :] % T2
_TMASK_E = (_ROW_T == 2 * _COL_T2).astype(np.float32)
_TMASK_O = (_ROW_T == 2 * _COL_T2 + 1).astype(np.float32)

_SPREAD = ((np.arange(G * NT2)[None, :] // NT2 == np.arange(G * T2)[:, None] // T2)
           & (np.arange(G * NT2)[None, :] % T2 == np.arange(G * T2)[:, None] % T2)
           ).astype(np.float32)


def _atan2_0_2pi(y, x):
    ax = jnp.abs(x)
    ay = jnp.abs(y)
    hi = jnp.maximum(ax, ay)
    lo = jnp.minimum(ax, ay)
    q = lo / jnp.where(hi == 0.0, 1.0, hi)
    big = q > 0.4142135623730950
    w = jnp.where(big, (q - 1.0) / (q + 1.0), q)
    z = w * w
    r = (((8.05374449538e-2 * z - 1.38776856032e-1) * z
          + 1.99777106478e-1) * z - 3.33329491539e-1) * z * w + w
    r = jnp.where(big, QUARTER_PI + r, r)
    r = jnp.where(ay > ax, HALF_PI - r, r)
    r = jnp.where(x < 0.0, PI - r, r)
    r = jnp.where(y < 0.0, -r, r)
    return jnp.where(r < 0.0, r + TWO_PI, r)


def _bucketize(px, py):
    dist = jnp.sqrt(px * px + py * py)
    ang = _atan2_0_2pi(px, py)
    valid = ((jnp.abs(px) + jnp.abs(py)) != 0.0) & (dist > 0.005)
    pidx = jnp.where(valid, (ang * INV_BUCKET).astype(jnp.int32), -1)
    return dist, ang, pidx


def _fused_kernel(xn4_ref, xe4_ref, spread_ref,
                  f_ego_ref, f_nei_ref, tme_ref, tmo_ref,
                  W12_ref, b12_ref, W22_ref, b22_ref, W32_ref, b32_ref,
                  Wce_ref, bce_ref, out_ref):
    fn = f_nei_ref[...].reshape(G, N, KC, 16, 128)
    fe = jnp.broadcast_to(
        f_ego_ref[...].reshape(G, 1, KC, 16, 128),
        (G, N, KC, 16, 128))
    f = jnp.max(fn * fe, axis=2)
    h = f.reshape(G * N * 16, 128)
    h = jax.nn.relu(jnp.dot(h, W12_ref[...],
                            preferred_element_type=jnp.float32) + b12_ref[...])
    h = jax.nn.relu(jnp.dot(h, W22_ref[...],
                            preferred_element_type=jnp.float32) + b22_ref[...])
    f2 = jax.nn.relu(jnp.dot(h, W32_ref[...],
                             preferred_element_type=jnp.float32) + b32_ref[...])

    vt = jnp.transpose(xn4_ref[0], (1, 0))
    et = jnp.transpose(xe4_ref[0], (1, 0))
    ego = jnp.dot(et, spread_ref[...],
                  preferred_element_type=jnp.float32)
    rel = vt - ego
    dist_e, ang_e, pidx_e = _bucketize(rel[0:1, :], rel[1:2, :])
    dist_o, ang_o, pidx_o = _bucketize(rel[2:3, :], rel[3:4, :])

    tme = tme_ref[...]
    tmo = tmo_ref[...]
    p_of_row = jax.lax.broadcasted_iota(jnp.int32, (TP, 1), 0) % P
    wce0 = Wce_ref[0:1, :]
    wce1 = Wce_ref[1:2, :]
    bce = bce_ref[...]
    ones_col = jnp.ones((NT2, 1), jnp.float32)
    y_rows = []
    for g in range(G):
        sl = slice(g * NT2, (g + 1) * NT2)
        f2g = f2[g * NT2:(g + 1) * NT2, :]
        m2e = jnp.where(pidx_e[:, sl] == p_of_row, tme, 0.0)
        m2o = jnp.where(pidx_o[:, sl] == p_of_row, tmo, 0.0)
        aux_e = jnp.concatenate([dist_e[:, sl], ang_e[:, sl]], axis=0)
        aux_o = jnp.concatenate([dist_o[:, sl], ang_o[:, sl]], axis=0)
        rhs_e = jnp.concatenate([f2g[:, 0:D2], aux_e.T, ones_col], axis=1)
        rhs_o = jnp.concatenate([f2g[:, D2:D], aux_o.T, ones_col], axis=1)
        out_g = (jnp.dot(m2e, rhs_e, preferred_element_type=jnp.float32)
                 + jnp.dot(m2o, rhs_o, preferred_element_type=jnp.float32))
        inv = 1.0 / (out_g[:, D2 + 2:D2 + 3] + 0.0001)
        ys = out_g[:, 0:D2] * inv
        pd = out_g[:, D2:D2 + 1] * inv
        pa = out_g[:, D2 + 1:D2 + 2] * inv
        pe = jax.nn.relu(pd * wce0 + pa * wce1 + bce)
        y_rows.append(jnp.concatenate([ys, pe], axis=1).reshape(T, P, D))
    out_ref[...] = jnp.stack(y_rows, axis=0)


def _blockdiag(W):
    z = jnp.zeros_like(W)
    return jnp.concatenate(
        [jnp.concatenate([W, z], axis=1), jnp.concatenate([z, W], axis=1)],
        axis=0)


@jax.jit
def kernel(x_ego_mean, x_nei_mean, f_ego, f_nei, W1, b1, W2, b2, W3, b3,
           Wce, bce):
    xn4 = x_nei_mean.reshape(B // G, G * N * T2, 4)
    xe4 = x_ego_mean.reshape(B // G, G * T2, 4)
    spread = jnp.asarray(_SPREAD)
    f_nei_v = f_nei.reshape(B, N * KC * 16, 128)
    f_ego_v = f_ego.reshape(B, KC * 16, 128)
    tme = jnp.asarray(_TMASK_E)
    tmo = jnp.asarray(_TMASK_O)
    W12 = _blockdiag(W1)
    W22 = _blockdiag(W2)
    W32 = _blockdiag(W3)
    b12 = jnp.concatenate([b1, b1]).reshape(1, 2 * DH)
    b22 = jnp.concatenate([b2, b2]).reshape(1, 2 * DH)
    b32 = jnp.concatenate([b3, b3]).reshape(1, D)
    bce = bce.reshape(1, D2)

    full = lambda shape: pl.BlockSpec(shape, lambda i: (0,) * len(shape))
    out = pl.pallas_call(
        _fused_kernel,
        grid=(B // G,),
        in_specs=[
            pl.BlockSpec((1, G * N * T2, 4), lambda i: (i, 0, 0)),
            pl.BlockSpec((1, G * T2, 4), lambda i: (i, 0, 0)),
            full((G * T2, G * NT2)),
            pl.BlockSpec((G, KC * 16, 128), lambda i: (i, 0, 0)),
            pl.BlockSpec((G, N * KC * 16, 128), lambda i: (i, 0, 0)),
            full((TP, NT2)), full((TP, NT2)),
            full((2 * DH, 2 * DH)), full((1, 2 * DH)),
            full((2 * DH, 2 * DH)), full((1, 2 * DH)),
            full((2 * DH, D)), full((1, D)),
            full((2, D2)), full((1, D2)),
        ],
        out_specs=pl.BlockSpec((G, T, P, D), lambda i: (i, 0, 0, 0)),
        out_shape=jax.ShapeDtypeStruct((B, T, P, D), jnp.float32),
    )(xn4, xe4, spread, f_ego_v, f_nei_v, tme, tmo,
      W12, b12, W22, b22, W32, b32, Wce, bce)
    return out


# speedup vs baseline: 1.0820x; 1.0820x over previous
import jax
import jax.numpy as jnp
import numpy as np
from jax.experimental import pallas as pl

B, N, KC, T, D, DH, P = 128, 64, 2, 32, 64, 64, 8
D2 = D // 2
G = 4
T2 = T // 2
NT2 = N * T2
TP = T * P
TWO_PI = np.float32(2.0 * np.pi)
PI = np.float32(np.pi)
HALF_PI = np.float32(np.pi / 2)
QUARTER_PI = np.float32(np.pi / 4)
INV_BUCKET = np.float32(P / (2.0 * np.pi))

_ROW_T = np.arange(TP)[:, None] // P
_COL_T2 = np.arange(NT2)[None, :] % T2
_TMASK_E = (_ROW_T == 2 * _COL_T2).astype(np.float32)
_TMASK_O = (_ROW_T == 2 * _COL_T2 + 1).astype(np.float32)

_SPREAD = ((np.arange(G * NT2)[None, :] // NT2 == np.arange(G * T2)[:, None] // T2)
           & (np.arange(G * NT2)[None, :] % T2 == np.arange(G * T2)[:, None] % T2)
           ).astype(np.float32)


def _atan2_0_2pi(y, x):
    ax = jnp.abs(x)
    ay = jnp.abs(y)
    hi = jnp.maximum(ax, ay)
    lo = jnp.minimum(ax, ay)
    q = lo / jnp.where(hi == 0.0, 1.0, hi)
    big = q > 0.4142135623730950
    w = jnp.where(big, (q - 1.0) / (q + 1.0), q)
    z = w * w
    r = (((8.05374449538e-2 * z - 1.38776856032e-1) * z
          + 1.99777106478e-1) * z - 3.33329491539e-1) * z * w + w
    r = jnp.where(big, QUARTER_PI + r, r)
    r = jnp.where(ay > ax, HALF_PI - r, r)
    r = jnp.where(x < 0.0, PI - r, r)
    r = jnp.where(y < 0.0, -r, r)
    return jnp.where(r < 0.0, r + TWO_PI, r)


def _bucketize(px, py):
    dist = jnp.sqrt(px * px + py * py)
    ang = _atan2_0_2pi(px, py)
    valid = ((jnp.abs(px) + jnp.abs(py)) != 0.0) & (dist > 0.005)
    pidx = jnp.where(valid, (ang * INV_BUCKET).astype(jnp.int32), -1)
    return dist, ang, pidx


def _fused_kernel(xn4_ref, xe4_ref, spread_ref,
                  f_ego_ref, f_nei_ref, tme_ref, tmo_ref,
                  W12_ref, b12_ref, W22_ref, b22_ref, W32_ref, b32_ref,
                  Wce_ref, bce_ref, out_ref):
    fn = f_nei_ref[...].reshape(G, N, KC, 16, 128)
    fe = jnp.broadcast_to(
        f_ego_ref[...].reshape(G, 1, KC, 16, 128),
        (G, N, KC, 16, 128))
    f = jnp.max(fn * fe, axis=2)
    h = f.reshape(G * N * 16, 128)
    h = jax.nn.relu(jnp.dot(h, W12_ref[...],
                            preferred_element_type=jnp.float32) + b12_ref[...])
    h = jax.nn.relu(jnp.dot(h, W22_ref[...],
                            preferred_element_type=jnp.float32) + b22_ref[...])
    f2 = jax.nn.relu(jnp.dot(h, W32_ref[...],
                             preferred_element_type=jnp.float32) + b32_ref[...])

    eye4 = jnp.float32(1.0) * (
        jax.lax.broadcasted_iota(jnp.int32, (4, 4), 0)
        == jax.lax.broadcasted_iota(jnp.int32, (4, 4), 1))
    vt = jax.lax.dot_general(eye4, xn4_ref[0], (((1,), (1,)), ((), ())),
                             preferred_element_type=jnp.float32)
    et = jax.lax.dot_general(eye4, xe4_ref[0], (((1,), (1,)), ((), ())),
                             preferred_element_type=jnp.float32)
    ego = jnp.dot(et, spread_ref[...],
                  preferred_element_type=jnp.float32)
    rel = vt - ego
    dist_e, ang_e, pidx_e = _bucketize(rel[0:1, :], rel[1:2, :])
    dist_o, ang_o, pidx_o = _bucketize(rel[2:3, :], rel[3:4, :])

    tme = tme_ref[...]
    tmo = tmo_ref[...]
    p_of_row = jax.lax.broadcasted_iota(jnp.int32, (TP, 1), 0) % P
    wce0 = Wce_ref[0:1, :]
    wce1 = Wce_ref[1:2, :]
    bce = bce_ref[...]
    ones_col = jnp.ones((NT2, 1), jnp.float32)
    y_rows = []
    for g in range(G):
        sl = slice(g * NT2, (g + 1) * NT2)
        f2g = f2[g * NT2:(g + 1) * NT2, :]
        m2e = jnp.where(pidx_e[:, sl] == p_of_row, tme, 0.0)
        m2o = jnp.where(pidx_o[:, sl] == p_of_row, tmo, 0.0)
        aux_e = jnp.concatenate([dist_e[:, sl], ang_e[:, sl]], axis=0)
        aux_o = jnp.concatenate([dist_o[:, sl], ang_o[:, sl]], axis=0)
        rhs_e = jnp.concatenate([f2g[:, 0:D2], aux_e.T, ones_col], axis=1)
        rhs_o = jnp.concatenate([f2g[:, D2:D], aux_o.T, ones_col], axis=1)
        out_g = (jnp.dot(m2e, rhs_e, preferred_element_type=jnp.float32)
                 + jnp.dot(m2o, rhs_o, preferred_element_type=jnp.float32))
        inv = 1.0 / (out_g[:, D2 + 2:D2 + 3] + 0.0001)
        ys = out_g[:, 0:D2] * inv
        pd = out_g[:, D2:D2 + 1] * inv
        pa = out_g[:, D2 + 1:D2 + 2] * inv
        pe = jax.nn.relu(pd * wce0 + pa * wce1 + bce)
        y_rows.append(jnp.concatenate([ys, pe], axis=1).reshape(T, P, D))
    out_ref[...] = jnp.stack(y_rows, axis=0)


def _blockdiag(W):
    z = jnp.zeros_like(W)
    return jnp.concatenate(
        [jnp.concatenate([W, z], axis=1), jnp.concatenate([z, W], axis=1)],
        axis=0)


@jax.jit
def kernel(x_ego_mean, x_nei_mean, f_ego, f_nei, W1, b1, W2, b2, W3, b3,
           Wce, bce):
    xn4 = x_nei_mean.reshape(B // G, G * N * T2, 4)
    xe4 = x_ego_mean.reshape(B // G, G * T2, 4)
    spread = jnp.asarray(_SPREAD)
    f_nei_v = f_nei.reshape(B, N * KC * 16, 128)
    f_ego_v = f_ego.reshape(B, KC * 16, 128)
    tme = jnp.asarray(_TMASK_E)
    tmo = jnp.asarray(_TMASK_O)
    W12 = _blockdiag(W1)
    W22 = _blockdiag(W2)
    W32 = _blockdiag(W3)
    b12 = jnp.concatenate([b1, b1]).reshape(1, 2 * DH)
    b22 = jnp.concatenate([b2, b2]).reshape(1, 2 * DH)
    b32 = jnp.concatenate([b3, b3]).reshape(1, D)
    bce = bce.reshape(1, D2)

    full = lambda shape: pl.BlockSpec(shape, lambda i: (0,) * len(shape))
    out = pl.pallas_call(
        _fused_kernel,
        grid=(B // G,),
        in_specs=[
            pl.BlockSpec((1, G * N * T2, 4), lambda i: (i, 0, 0)),
            pl.BlockSpec((1, G * T2, 4), lambda i: (i, 0, 0)),
            full((G * T2, G * NT2)),
            pl.BlockSpec((G, KC * 16, 128), lambda i: (i, 0, 0)),
            pl.BlockSpec((G, N * KC * 16, 128), lambda i: (i, 0, 0)),
            full((TP, NT2)), full((TP, NT2)),
            full((2 * DH, 2 * DH)), full((1, 2 * DH)),
            full((2 * DH, 2 * DH)), full((1, 2 * DH)),
            full((2 * DH, D)), full((1, D)),
            full((2, D2)), full((1, D2)),
        ],
        out_specs=pl.BlockSpec((G, T, P, D), lambda i: (i, 0, 0, 0)),
        out_shape=jax.ShapeDtypeStruct((B, T, P, D), jnp.float32),
    )(xn4, xe4, spread, f_ego_v, f_nei_v, tme, tmo,
      W12, b12, W22, b22, W32, b32, Wce, bce)
    return out


# speedup vs baseline: 1.5916x; 1.4710x over previous
import jax
import jax.numpy as jnp
import numpy as np
from jax.experimental import pallas as pl

B, N, KC, T, D, DH, P = 128, 64, 2, 32, 64, 64, 8
D2 = D // 2
G = 4
T2 = T // 2
NT2 = N * T2
TP = T * P
TWO_PI = np.float32(2.0 * np.pi)
PI = np.float32(np.pi)
HALF_PI = np.float32(np.pi / 2)
QUARTER_PI = np.float32(np.pi / 4)
INV_BUCKET = np.float32(P / (2.0 * np.pi))

_ROW_T = np.arange(TP)[:, None] // P
_COL_T2 = np.arange(NT2)[None, :] % T2
_TMASK_E = (_ROW_T == 2 * _COL_T2).astype(np.float32)
_TMASK_O = (_ROW_T == 2 * _COL_T2 + 1).astype(np.float32)

_SPREAD = ((np.arange(G * NT2)[None, :] // NT2 == np.arange(G * T2)[:, None] // T2)
           & (np.arange(G * NT2)[None, :] % T2 == np.arange(G * T2)[:, None] % T2)
           ).astype(np.float32)


def _atan2_0_2pi(y, x):
    ax = jnp.abs(x)
    ay = jnp.abs(y)
    hi = jnp.maximum(ax, ay)
    lo = jnp.minimum(ax, ay)
    q = lo / jnp.where(hi == 0.0, 1.0, hi)
    big = q > 0.4142135623730950
    w = jnp.where(big, (q - 1.0) / (q + 1.0), q)
    z = w * w
    r = (((8.05374449538e-2 * z - 1.38776856032e-1) * z
          + 1.99777106478e-1) * z - 3.33329491539e-1) * z * w + w
    r = jnp.where(big, QUARTER_PI + r, r)
    r = jnp.where(ay > ax, HALF_PI - r, r)
    r = jnp.where(x < 0.0, PI - r, r)
    r = jnp.where(y < 0.0, -r, r)
    return jnp.where(r < 0.0, r + TWO_PI, r)


def _bucketize(px_ref, py_ref, ex_ref, ey_ref):
    ex = jnp.concatenate([ex_ref[0]] * N, axis=1)
    ey = jnp.concatenate([ey_ref[0]] * N, axis=1)
    px = px_ref[0] - ex
    py = py_ref[0] - ey
    dist = jnp.sqrt(px * px + py * py)
    ang = _atan2_0_2pi(px, py)
    valid = ((jnp.abs(px) + jnp.abs(py)) != 0.0) & (dist > 0.005)
    pidx = jnp.where(valid, (ang * INV_BUCKET).astype(jnp.int32), -1)
    return dist, ang, pidx


def _fused_kernel(pxe_ref, pye_ref, pxo_ref, pyo_ref,
                  exe_ref, eye_ref, exo_ref, eyo_ref,
                  f_ego_ref, f_nei_ref, tme_ref, tmo_ref,
                  W12_ref, b12_ref, W22_ref, b22_ref, W32_ref, b32_ref,
                  Wce_ref, bce_ref, out_ref):
    fn = f_nei_ref[...].reshape(G, N, KC, 16, 128)
    fe = jnp.broadcast_to(
        f_ego_ref[...].reshape(G, 1, KC, 16, 128),
        (G, N, KC, 16, 128))
    f = jnp.max(fn * fe, axis=2)
    h = f.reshape(G * N * 16, 128)
    h = jax.nn.relu(jnp.dot(h, W12_ref[...],
                            preferred_element_type=jnp.float32) + b12_ref[...])
    h = jax.nn.relu(jnp.dot(h, W22_ref[...],
                            preferred_element_type=jnp.float32) + b22_ref[...])
    f2 = jax.nn.relu(jnp.dot(h, W32_ref[...],
                             preferred_element_type=jnp.float32) + b32_ref[...])

    dist_e, ang_e, pidx_e = _bucketize(pxe_ref, pye_ref, exe_ref, eye_ref)
    dist_o, ang_o, pidx_o = _bucketize(pxo_ref, pyo_ref, exo_ref, eyo_ref)

    tme = tme_ref[...]
    tmo = tmo_ref[...]
    p_of_row = jax.lax.broadcasted_iota(jnp.int32, (TP, 1), 0) % P
    wce0 = Wce_ref[0:1, :]
    wce1 = Wce_ref[1:2, :]
    bce = bce_ref[...]
    ones_col = jnp.ones((NT2, 1), jnp.float32)
    y_rows = []
    for g in range(G):
        f2g = f2[g * NT2:(g + 1) * NT2, :]
        m2e = jnp.where(pidx_e[g:g + 1, :] == p_of_row, tme, 0.0)
        m2o = jnp.where(pidx_o[g:g + 1, :] == p_of_row, tmo, 0.0)
        aux_e = jnp.concatenate([dist_e[g:g + 1, :], ang_e[g:g + 1, :]], axis=0)
        aux_o = jnp.concatenate([dist_o[g:g + 1, :], ang_o[g:g + 1, :]], axis=0)
        rhs_e = jnp.concatenate([f2g[:, 0:D2], aux_e.T, ones_col], axis=1)
        rhs_o = jnp.concatenate([f2g[:, D2:D], aux_o.T, ones_col], axis=1)
        out_g = (jnp.dot(m2e, rhs_e, preferred_element_type=jnp.float32)
                 + jnp.dot(m2o, rhs_o, preferred_element_type=jnp.float32))
        inv = 1.0 / (out_g[:, D2 + 2:D2 + 3] + 0.0001)
        ys = out_g[:, 0:D2] * inv
        pd = out_g[:, D2:D2 + 1] * inv
        pa = out_g[:, D2 + 1:D2 + 2] * inv
        pe = jax.nn.relu(pd * wce0 + pa * wce1 + bce)
        y_rows.append(jnp.concatenate([ys, pe], axis=1).reshape(T, P, D))
    out_ref[...] = jnp.stack(y_rows, axis=0)


def _blockdiag(W):
    z = jnp.zeros_like(W)
    return jnp.concatenate(
        [jnp.concatenate([W, z], axis=1), jnp.concatenate([z, W], axis=1)],
        axis=0)


@jax.jit
def kernel(x_ego_mean, x_nei_mean, f_ego, f_nei, W1, b1, W2, b2, W3, b3,
           Wce, bce):
    pxe = x_nei_mean[:, :, 0::2, 0].reshape(B // G, G, NT2)
    pxo = x_nei_mean[:, :, 1::2, 0].reshape(B // G, G, NT2)
    pye = x_nei_mean[:, :, 0::2, 1].reshape(B // G, G, NT2)
    pyo = x_nei_mean[:, :, 1::2, 1].reshape(B // G, G, NT2)
    exe = x_ego_mean[:, 0::2, 0].reshape(B // G, G, T2)
    exo = x_ego_mean[:, 1::2, 0].reshape(B // G, G, T2)
    eye_ = x_ego_mean[:, 0::2, 1].reshape(B // G, G, T2)
    eyo = x_ego_mean[:, 1::2, 1].reshape(B // G, G, T2)
    f_nei_v = f_nei.reshape(B, N * KC * 16, 128)
    f_ego_v = f_ego.reshape(B, KC * 16, 128)
    tme = jnp.asarray(_TMASK_E)
    tmo = jnp.asarray(_TMASK_O)
    W12 = _blockdiag(W1)
    W22 = _blockdiag(W2)
    W32 = _blockdiag(W3)
    b12 = jnp.concatenate([b1, b1]).reshape(1, 2 * DH)
    b22 = jnp.concatenate([b2, b2]).reshape(1, 2 * DH)
    b32 = jnp.concatenate([b3, b3]).reshape(1, D)
    bce = bce.reshape(1, D2)

    full = lambda shape: pl.BlockSpec(shape, lambda i: (0,) * len(shape))
    out = pl.pallas_call(
        _fused_kernel,
        grid=(B // G,),
        in_specs=[
            pl.BlockSpec((1, G, NT2), lambda i: (i, 0, 0)),
            pl.BlockSpec((1, G, NT2), lambda i: (i, 0, 0)),
            pl.BlockSpec((1, G, NT2), lambda i: (i, 0, 0)),
            pl.BlockSpec((1, G, NT2), lambda i: (i, 0, 0)),
            pl.BlockSpec((1, G, T2), lambda i: (i, 0, 0)),
            pl.BlockSpec((1, G, T2), lambda i: (i, 0, 0)),
            pl.BlockSpec((1, G, T2), lambda i: (i, 0, 0)),
            pl.BlockSpec((1, G, T2), lambda i: (i, 0, 0)),
            pl.BlockSpec((G, KC * 16, 128), lambda i: (i, 0, 0)),
            pl.BlockSpec((G, N * KC * 16, 128), lambda i: (i, 0, 0)),
            full((TP, NT2)), full((TP, NT2)),
            full((2 * DH, 2 * DH)), full((1, 2 * DH)),
            full((2 * DH, 2 * DH)), full((1, 2 * DH)),
            full((2 * DH, D)), full((1, D)),
            full((2, D2)), full((1, D2)),
        ],
        out_specs=pl.BlockSpec((G, T, P, D), lambda i: (i, 0, 0, 0)),
        out_shape=jax.ShapeDtypeStruct((B, T, P, D), jnp.float32),
    )(pxe, pye, pxo, pyo, exe, eye_, exo, eyo, f_ego_v, f_nei_v, tme, tmo,
      W12, b12, W22, b22, W32, b32, Wce, bce)
    return out


# speedup vs baseline: 1.7068x; 1.0724x over previous
import jax
import jax.numpy as jnp
import numpy as np
from jax.experimental import pallas as pl

B, N, KC, T, D, DH, P = 128, 64, 2, 32, 64, 64, 8
D2 = D // 2
G = 4
T2 = T // 2
NT2 = N * T2
TP = T * P
TWO_PI = np.float32(2.0 * np.pi)
PI = np.float32(np.pi)
HALF_PI = np.float32(np.pi / 2)
QUARTER_PI = np.float32(np.pi / 4)
INV_BUCKET = np.float32(P / (2.0 * np.pi))

_ROW_T = np.arange(TP)[:, None] // P
_COL_T2 = np.arange(NT2)[None, :] % T2
_TMASK_E = (_ROW_T == 2 * _COL_T2).astype(np.float32)
_TMASK_O = (_ROW_T == 2 * _COL_T2 + 1).astype(np.float32)

_SPREAD = ((np.arange(G * NT2)[None, :] // NT2 == np.arange(G * T2)[:, None] // T2)
           & (np.arange(G * NT2)[None, :] % T2 == np.arange(G * T2)[:, None] % T2)
           ).astype(np.float32)


def _atan2_0_2pi(y, x):
    ax = jnp.abs(x)
    ay = jnp.abs(y)
    hi = jnp.maximum(ax, ay)
    lo = jnp.minimum(ax, ay)
    q = lo / jnp.where(hi == 0.0, 1.0, hi)
    big = q > 0.4142135623730950
    w = jnp.where(big, (q - 1.0) / (q + 1.0), q)
    z = w * w
    r = (((8.05374449538e-2 * z - 1.38776856032e-1) * z
          + 1.99777106478e-1) * z - 3.33329491539e-1) * z * w + w
    r = jnp.where(big, QUARTER_PI + r, r)
    r = jnp.where(ay > ax, HALF_PI - r, r)
    r = jnp.where(x < 0.0, PI - r, r)
    r = jnp.where(y < 0.0, -r, r)
    return jnp.where(r < 0.0, r + TWO_PI, r)


def _bucketize(px_ref, py_ref, ex_ref, ey_ref):
    ex = jnp.concatenate([ex_ref[0]] * N, axis=1)
    ey = jnp.concatenate([ey_ref[0]] * N, axis=1)
    px = px_ref[0] - ex
    py = py_ref[0] - ey
    dist = jnp.sqrt(px * px + py * py)
    ang = _atan2_0_2pi(px, py)
    valid = ((jnp.abs(px) + jnp.abs(py)) != 0.0) & (dist > 0.005)
    pidx = jnp.where(valid, (ang * INV_BUCKET).astype(jnp.int32), -1)
    return dist, ang, pidx


def _fused_kernel(pxe_ref, pye_ref, pxo_ref, pyo_ref,
                  exe_ref, eye_ref, exo_ref, eyo_ref,
                  f_ego_ref, f_nei_ref, tme_ref, tmo_ref,
                  W12_ref, b12_ref, W22_ref, b22_ref, W32_ref, b32_ref,
                  Wce_ref, bce_ref, out_ref):
    fn = f_nei_ref[...].reshape(G, N, KC, 16, 128)
    fe = jnp.broadcast_to(
        f_ego_ref[...].reshape(G, 1, KC, 16, 128),
        (G, N, KC, 16, 128))
    f = jnp.max(fn * fe, axis=2)
    h = f.reshape(G * N * 16, 128)
    h = jax.nn.relu(jnp.dot(h, W12_ref[...],
                            preferred_element_type=jnp.float32) + b12_ref[...])
    h = jax.nn.relu(jnp.dot(h, W22_ref[...],
                            preferred_element_type=jnp.float32) + b22_ref[...])
    f2 = jax.nn.relu(jnp.dot(h, W32_ref[...],
                             preferred_element_type=jnp.float32) + b32_ref[...])

    dist_e, ang_e, pidx_e = _bucketize(pxe_ref, pye_ref, exe_ref, eye_ref)
    dist_o, ang_o, pidx_o = _bucketize(pxo_ref, pyo_ref, exo_ref, eyo_ref)

    tme = tme_ref[...]
    tmo = tmo_ref[...]
    p_of_row = jax.lax.broadcasted_iota(jnp.int32, (TP, 1), 0) % P
    wce0 = Wce_ref[0:1, :]
    wce1 = Wce_ref[1:2, :]
    bce = bce_ref[...]
    ones_row = jnp.ones((1, NT2), jnp.float32)
    dnums = (((1,), (1,)), ((), ()))
    y_rows = []
    for g in range(G):
        f2g = f2[g * NT2:(g + 1) * NT2, :]
        m2e = jnp.where(pidx_e[g:g + 1, :] == p_of_row, tme, 0.0)
        m2o = jnp.where(pidx_o[g:g + 1, :] == p_of_row, tmo, 0.0)
        aux_e = jnp.concatenate(
            [dist_e[g:g + 1, :], ang_e[g:g + 1, :], ones_row], axis=0)
        aux_o = jnp.concatenate(
            [dist_o[g:g + 1, :], ang_o[g:g + 1, :], ones_row], axis=0)
        ys_g = (jnp.dot(m2e, f2g[:, 0:D2], preferred_element_type=jnp.float32)
                + jnp.dot(m2o, f2g[:, D2:D], preferred_element_type=jnp.float32))
        pos_g = (jax.lax.dot_general(m2e, aux_e, dnums,
                                     preferred_element_type=jnp.float32)
                 + jax.lax.dot_general(m2o, aux_o, dnums,
                                       preferred_element_type=jnp.float32))
        inv = 1.0 / (pos_g[:, 2:3] + 0.0001)
        ys = ys_g * inv
        pd = pos_g[:, 0:1] * inv
        pa = pos_g[:, 1:2] * inv
        pe = jax.nn.relu(pd * wce0 + pa * wce1 + bce)
        y_rows.append(jnp.concatenate([ys, pe], axis=1).reshape(T, P, D))
    out_ref[...] = jnp.stack(y_rows, axis=0)


def _blockdiag(W):
    z = jnp.zeros_like(W)
    return jnp.concatenate(
        [jnp.concatenate([W, z], axis=1), jnp.concatenate([z, W], axis=1)],
        axis=0)


@jax.jit
def kernel(x_ego_mean, x_nei_mean, f_ego, f_nei, W1, b1, W2, b2, W3, b3,
           Wce, bce):
    pxe = x_nei_mean[:, :, 0::2, 0].reshape(B // G, G, NT2)
    pxo = x_nei_mean[:, :, 1::2, 0].reshape(B // G, G, NT2)
    pye = x_nei_mean[:, :, 0::2, 1].reshape(B // G, G, NT2)
    pyo = x_nei_mean[:, :, 1::2, 1].reshape(B // G, G, NT2)
    exe = x_ego_mean[:, 0::2, 0].reshape(B // G, G, T2)
    exo = x_ego_mean[:, 1::2, 0].reshape(B // G, G, T2)
    eye_ = x_ego_mean[:, 0::2, 1].reshape(B // G, G, T2)
    eyo = x_ego_mean[:, 1::2, 1].reshape(B // G, G, T2)
    f_nei_v = f_nei.reshape(B, N * KC * 16, 128)
    f_ego_v = f_ego.reshape(B, KC * 16, 128)
    tme = jnp.asarray(_TMASK_E)
    tmo = jnp.asarray(_TMASK_O)
    W12 = _blockdiag(W1)
    W22 = _blockdiag(W2)
    W32 = _blockdiag(W3)
    b12 = jnp.concatenate([b1, b1]).reshape(1, 2 * DH)
    b22 = jnp.concatenate([b2, b2]).reshape(1, 2 * DH)
    b32 = jnp.concatenate([b3, b3]).reshape(1, D)
    bce = bce.reshape(1, D2)

    full = lambda shape: pl.BlockSpec(shape, lambda i: (0,) * len(shape))
    out = pl.pallas_call(
        _fused_kernel,
        grid=(B // G,),
        in_specs=[
            pl.BlockSpec((1, G, NT2), lambda i: (i, 0, 0)),
            pl.BlockSpec((1, G, NT2), lambda i: (i, 0, 0)),
            pl.BlockSpec((1, G, NT2), lambda i: (i, 0, 0)),
            pl.BlockSpec((1, G, NT2), lambda i: (i, 0, 0)),
            pl.BlockSpec((1, G, T2), lambda i: (i, 0, 0)),
            pl.BlockSpec((1, G, T2), lambda i: (i, 0, 0)),
            pl.BlockSpec((1, G, T2), lambda i: (i, 0, 0)),
            pl.BlockSpec((1, G, T2), lambda i: (i, 0, 0)),
            pl.BlockSpec((G, KC * 16, 128), lambda i: (i, 0, 0)),
            pl.BlockSpec((G, N * KC * 16, 128), lambda i: (i, 0, 0)),
            full((TP, NT2)), full((TP, NT2)),
            full((2 * DH, 2 * DH)), full((1, 2 * DH)),
            full((2 * DH, 2 * DH)), full((1, 2 * DH)),
            full((2 * DH, D)), full((1, D)),
            full((2, D2)), full((1, D2)),
        ],
        out_specs=pl.BlockSpec((G, T, P, D), lambda i: (i, 0, 0, 0)),
        out_shape=jax.ShapeDtypeStruct((B, T, P, D), jnp.float32),
    )(pxe, pye, pxo, pyo, exe, eye_, exo, eyo, f_ego_v, f_nei_v, tme, tmo,
      W12, b12, W22, b22, W32, b32, Wce, bce)
    return out
